# Initial kernel scaffold; baseline (speedup 1.0000x reference)
import jax
import jax.numpy as jnp
from jax.experimental import pallas as pl


def kernel(x, length, conv_w, conv_b):
    raise NotImplementedError("write your pallas kernel here")


# speedup vs baseline: 6.5478x; 6.5478x over previous
import functools

import jax
import jax.numpy as jnp
import numpy as np
from jax import lax
from jax.experimental import pallas as pl
from jax.experimental.pallas import tpu as pltpu
from jax.experimental.pallas import tpu_sc as plsc

N = 65536
D = 32
NSEG = 16
TOPK = 1024
LANES = 16
WIN = 4160
NV = WIN // LANES
HB = 256
TIE_B = 64
IMIN = np.int32(-2147483648)


def _sortable_i32(bits):
    return jnp.where(bits >= 0, bits, bits ^ np.int32(0x7FFFFFFF))


def _att_body(x_ref, w_ref, b_ref, o_ref):
    xb = x_ref[...]
    w = w_ref[...]
    att = lax.dot_general(xb, w, (((1,), (0,)), ((), ())),
                          preferred_element_type=jnp.float32)
    att = att.reshape(1, 1, 4096) + b_ref[0, 0]
    o_ref[...] = att


def _compute_att(x, w2d, b2d):
    return pl.pallas_call(
        _att_body,
        grid=(NSEG,),
        in_specs=[
            pl.BlockSpec((N // NSEG, D), lambda i: (i, 0)),
            pl.BlockSpec((D, 1), lambda i: (0, 0)),
            pl.BlockSpec((1, 1), lambda i: (0, 0)),
        ],
        out_specs=pl.BlockSpec((1, 1, N // NSEG), lambda i: (i, 0, 0)),
        out_shape=jax.ShapeDtypeStruct((NSEG, 1, N // NSEG), jnp.float32),
    )(x, w2d, b2d)


@functools.lru_cache(maxsize=None)
def _get_sc_kernel():
    mesh = plsc.VectorSubcoreMesh(core_axis_name="c", subcore_axis_name="s")
    return functools.partial(
        pl.kernel,
        out_type=jax.ShapeDtypeStruct((NSEG, LANES), jnp.int32),
        mesh=mesh,
        scratch_types=[
            pltpu.VMEM((WIN,), jnp.float32),
            pltpu.VMEM((WIN,), jnp.int32),
            pltpu.VMEM((HB * LANES,), jnp.int32),
            pltpu.VMEM((LANES,), jnp.int32),
            pltpu.VMEM((LANES,), jnp.int32),
            pltpu.VMEM((LANES,), jnp.int32),
        ],
        compiler_params=pltpu.CompilerParams(needs_layout_passes=False),
    )(_sc_body)


def _at_lane(v, pos, lane):
    return jnp.sum(jnp.where(lane == pos, v, 0), axis=0)


def _scalar(v):
    return jnp.max(v, axis=0)


def _sc_body(att_hbm, starts_hbm, lens_hbm, out_hbm, win_v, keys_v, hist_v,
             sv_v, lv_v, res_v):
    lane = lax.iota(jnp.int32, LANES)
    cid = lax.axis_index("c")
    sid = lax.axis_index("s")
    wid = sid * 2 + cid

    @pl.when(wid < NSEG)
    def _():
        seg = wid
        pltpu.sync_copy(starts_hbm, sv_v)
        pltpu.sync_copy(lens_hbm, lv_v)
        start = _at_lane(sv_v[...], seg, lane)
        seglen = _at_lane(lv_v[...], seg, lane)

        a = pl.multiple_of(jnp.minimum(start & np.int32(-8), np.int32(N - WIN)), 8)
        off = start - a
        pltpu.sync_copy(att_hbm.at[pl.ds(a, WIN)], win_v)

        def conv_body(j, _):
            v = win_v[pl.ds(j * LANES, LANES)]
            bits = lax.bitcast_convert_type(v, jnp.int32)
            s = _sortable_i32(bits)
            lidx = j * LANES + lane - off
            valid = (lidx >= 0) & (lidx < seglen)
            keys_v[pl.ds(j * LANES, LANES)] = jnp.where(valid, s, IMIN)
            return 0

        lax.fori_loop(0, NV, conv_body, 0, unroll=4)

        zeros = lane & 0
        ones = zeros + 1

        def zero_hist(nwords):
            def zbody(k, _):
                hist_v[pl.ds(k * LANES, LANES)] = zeros
                return 0
            lax.fori_loop(0, nwords // LANES, zbody, 0, unroll=4)

        def key_level(shift, himask, prefix, remaining):
            zero_hist(HB * LANES)

            def hbody(j, _):
                s = keys_v[pl.ds(j * LANES, LANES)]
                u = s ^ IMIN
                match = (u & himask) == prefix
                bins = lax.shift_right_logical(u, shift) & np.int32(HB - 1)
                plsc.addupdate_scatter(hist_v, [bins * LANES + lane], ones,
                                       mask=match)
                return 0

            lax.fori_loop(0, NV, hbody, 0, unroll=4)

            def fbody(c, carry):
                found, b, rem, tot = carry
                cc = (HB // LANES - 1) - c
                base = (cc * LANES + lane) * LANES
                acc = zeros
                for l in range(LANES):
                    acc = acc + plsc.load_gather(hist_v, [base + l])
                rv = lax.rev(acc, (0,))
                cs = plsc.cumsum(rv)
                suf = tot + cs
                cond = suf >= rem
                anyc = _scalar(plsc.all_reduce_population_count(cond))
                ffs = plsc.all_reduce_ffs(cond)
                hit = jnp.logical_and(jnp.logical_not(found), anyc > 0)
                b_new = cc * LANES + (LANES - 1) - _scalar(ffs)
                s_at = _at_lane(suf, ffs, lane)
                h_at = _at_lane(rv, ffs, lane)
                above = s_at - h_at
                b = jnp.where(hit, b_new, b)
                rem = jnp.where(hit, rem - above, rem)
                found = jnp.logical_or(found, anyc > 0)
                tot = tot + _at_lane(cs, np.int32(LANES - 1), lane)
                return found, b, rem, tot

            init = (np.bool_(False), np.int32(0), remaining, np.int32(0))
            _, b, rem, _ = lax.fori_loop(0, HB // LANES, fbody, init)
            return b, rem

        def tie_level(nbits_shift, t_s, sel_hi, hi_shift, r):
            zero_hist(TIE_B * LANES)

            def hbody(j, _):
                s = keys_v[pl.ds(j * LANES, LANES)]
                lidx = j * LANES + lane - off
                bsel = lax.shift_right_logical(lidx, hi_shift)
                match = (s == t_s) & (bsel == sel_hi)
                bins = lax.shift_right_logical(lidx, nbits_shift) & np.int32(TIE_B - 1)
                plsc.addupdate_scatter(hist_v, [bins * LANES + lane], ones,
                                       mask=match)
                return 0

            lax.fori_loop(0, NV, hbody, 0, unroll=4)

            def fbody(c, carry):
                found, b, rem, tot = carry
                base = (c * LANES + lane) * LANES
                acc = zeros
                for l in range(LANES):
                    acc = acc + plsc.load_gather(hist_v, [base + l])
                cs = plsc.cumsum(acc)
                cum = tot + cs
                cond = cum >= rem
                anyc = _scalar(plsc.all_reduce_population_count(cond))
                ffs = plsc.all_reduce_ffs(cond)
                hit = jnp.logical_and(jnp.logical_not(found), anyc > 0)
                b_new = c * LANES + _scalar(ffs)
                c_at = _at_lane(cum, ffs, lane)
                h_at = _at_lane(acc, ffs, lane)
                below = c_at - h_at
                b = jnp.where(hit, b_new, b)
                rem = jnp.where(hit, rem - below, rem)
                found = jnp.logical_or(found, anyc > 0)
                tot = tot + _at_lane(cs, np.int32(LANES - 1), lane)
                return found, b, rem, tot

            init = (np.bool_(False), np.int32(0), r, np.int32(0))
            _, b, rem, _ = lax.fori_loop(0, TIE_B // LANES, fbody, init)
            return b, rem

        def full_search(_):
            prefix = np.int32(0)
            rem = np.int32(TOPK)
            himasks = (np.int32(0),
                       np.int32(-16777216),
                       np.int32(-65536),
                       np.int32(-256))
            for lvl, shift in enumerate((24, 16, 8, 0)):
                b, rem = key_level(np.int32(shift), himasks[lvl], prefix, rem)
                prefix = prefix | lax.shift_left(b, np.int32(shift))
            t_s = prefix ^ IMIN
            b0, rem = tie_level(np.int32(6), t_s, np.int32(0), np.int32(12), rem)
            b1, rem = tie_level(np.int32(0), t_s, b0, np.int32(6), rem)
            tie = lax.shift_left(b0, np.int32(6)) | b1
            return t_s, tie

        def trivial(_):
            return IMIN, np.int32(0)

        t_s, tie = lax.cond(seglen > TOPK, full_search, trivial, 0)

        res = jnp.where(lane == 0, t_s, jnp.where(lane == 1, tie, 0))
        res_v[...] = res
        pltpu.sync_copy(res_v, out_hbm.at[seg])


def _sc_thresholds(att_flat, starts, length):
    return _get_sc_kernel()(att_flat, starts, length)


def _pool_body(x_ref, att_ref, starts_ref, lens_ref, thr_ref, tie_ref,
               lenf_ref, o_ref, acc_ref):
    blk = pl.program_id(0)
    nblk = pl.num_programs(0)

    @pl.when(blk == 0)
    def _():
        acc_ref[...] = jnp.zeros((NSEG, D), jnp.float32)

    rows = lax.broadcasted_iota(jnp.int32, (NSEG, N // NSEG), 1) + blk * (N // NSEG)
    bits = lax.bitcast_convert_type(att_ref[...].reshape(1, N // NSEG),
                                    jnp.int32)
    key1 = _sortable_i32(bits)
    key = jnp.broadcast_to(key1, (NSEG, N // NSEG))
    starts = starts_ref[...]
    lens = lens_ref[...]
    thr = thr_ref[...]
    tie = tie_ref[...]
    in_seg = (rows >= starts) & (rows < starts + lens)
    loc = rows - starts
    sel = (key > thr) | ((key == thr) & (loc <= tie))
    w = (in_seg & sel).astype(jnp.float32)
    acc_ref[...] += lax.dot_general(
        w, x_ref[...], (((1,), (0,)), ((), ())),
        preferred_element_type=jnp.float32,
        precision=lax.Precision.HIGHEST)

    @pl.when(blk == nblk - 1)
    def _():
        res = acc_ref[...] / lenf_ref[...]
        nrm = jnp.sqrt(jnp.sum(res * res, axis=1, keepdims=True))
        o_ref[...] = res / jnp.maximum(nrm, 1e-12)


def _pool(x, att, starts2d, lens2d, thr2d, tie2d, lenf2d):
    full = lambda i: (0, 0)
    return pl.pallas_call(
        _pool_body,
        grid=(NSEG,),
        in_specs=[
            pl.BlockSpec((N // NSEG, D), lambda i: (i, 0)),
            pl.BlockSpec((1, 1, N // NSEG), lambda i: (i, 0, 0)),
            pl.BlockSpec((NSEG, 1), full),
            pl.BlockSpec((NSEG, 1), full),
            pl.BlockSpec((NSEG, 1), full),
            pl.BlockSpec((NSEG, 1), full),
            pl.BlockSpec((NSEG, 1), full),
        ],
        out_specs=pl.BlockSpec((NSEG, D), full),
        out_shape=jax.ShapeDtypeStruct((NSEG, D), jnp.float32),
        scratch_shapes=[pltpu.VMEM((NSEG, D), jnp.float32)],
    )(x, att, starts2d, lens2d, thr2d, tie2d, lenf2d)


def kernel(x, length, conv_w, conv_b):
    w2d = conv_w[0].astype(jnp.float32)
    b2d = conv_b.reshape(1, 1).astype(jnp.float32)
    length = length.astype(jnp.int32)
    starts = jnp.concatenate(
        [jnp.zeros((1,), jnp.int32), jnp.cumsum(length)[:-1]])

    att2d = _compute_att(x, w2d, b2d)
    att_flat = att2d.reshape(N)

    sc_out = _sc_thresholds(att_flat, starts, length)
    thr2d = sc_out[:, 0:1]
    tie2d = sc_out[:, 1:2]

    return _pool(x, att2d, starts.reshape(NSEG, 1), length.reshape(NSEG, 1),
                 thr2d, tie2d, length.astype(jnp.float32).reshape(NSEG, 1))


# speedup vs baseline: 14.1306x; 2.1581x over previous
import functools

import jax
import jax.numpy as jnp
import numpy as np
from jax import lax
from jax.experimental import pallas as pl
from jax.experimental.pallas import tpu as pltpu
from jax.experimental.pallas import tpu_sc as plsc

N = 65536
D = 32
NSEG = 16
TOPK = 1024
LANES = 16
WIN = 4160
NV = WIN // LANES
HB = 256
TIE_B = 64
NBLK = 8
BLK = N // NBLK
IMIN = np.int32(-2147483648)


def _sortable_i32(bits):
    return jnp.where(bits >= 0, bits, bits ^ np.int32(0x7FFFFFFF))


def _att_body(xt_ref, w_ref, b_ref, o_ref):
    xt = xt_ref[...]
    w = w_ref[...]
    att = lax.dot_general(w, xt, (((1,), (0,)), ((), ())),
                          preferred_element_type=jnp.float32)
    o_ref[...] = (att + b_ref[0, 0]).reshape(1, 1, BLK)


def _compute_att(xt, w2d, b2d):
    return pl.pallas_call(
        _att_body,
        grid=(NBLK,),
        in_specs=[
            pl.BlockSpec((D, BLK), lambda i: (0, i)),
            pl.BlockSpec((1, D), lambda i: (0, 0)),
            pl.BlockSpec((1, 1), lambda i: (0, 0)),
        ],
        out_specs=pl.BlockSpec((1, 1, BLK), lambda i: (i, 0, 0)),
        out_shape=jax.ShapeDtypeStruct((NBLK, 1, BLK), jnp.float32),
    )(xt, w2d, b2d)


SC_SCRATCH = (
    ((WIN,), jnp.float32),
    ((WIN,), jnp.int32),
    ((HB * LANES,), jnp.int32),
    ((WIN + 64,), jnp.int32),
    ((WIN + 64,), jnp.int32),
    ((LANES,), jnp.int32),
    ((LANES,), jnp.int32),
    ((LANES,), jnp.int32),
)


@functools.lru_cache(maxsize=None)
def _get_sc_kernel():
    mesh = plsc.VectorSubcoreMesh(core_axis_name="c", subcore_axis_name="s")
    return functools.partial(
        pl.kernel,
        out_type=jax.ShapeDtypeStruct((NSEG, LANES), jnp.int32),
        mesh=mesh,
        scratch_types=[pltpu.VMEM(s, d) for s, d in SC_SCRATCH],
        compiler_params=pltpu.CompilerParams(needs_layout_passes=False),
    )(_sc_body)


def _at_lane(v, pos, lane):
    return jnp.sum(jnp.where(lane == pos, v, 0), axis=0)


def _scalar(v):
    return jnp.max(v, axis=0)


def _sc_body(att_hbm, starts_hbm, lens_hbm, out_hbm, win_v, keys_v, hist_v,
             cu_v, cx_v, sv_v, lv_v, res_v):
    lane = lax.iota(jnp.int32, LANES)
    cid = lax.axis_index("c")
    sid = lax.axis_index("s")
    wid = sid * 2 + cid

    @pl.when(wid < NSEG)
    def _():
        seg = wid
        pltpu.sync_copy(starts_hbm, sv_v)
        pltpu.sync_copy(lens_hbm, lv_v)
        start = _at_lane(sv_v[...], seg, lane)
        seglen = _at_lane(lv_v[...], seg, lane)

        a = pl.multiple_of(jnp.minimum(start & np.int32(-8), np.int32(N - WIN)), 8)
        off = start - a
        nv4 = lax.shift_right_logical(off + seglen + np.int32(63), 6)

        zeros = lane & 0
        ones = zeros + 1

        def zero_hist(nwords):
            def zbody(k, _):
                hist_v[pl.ds(k * LANES, LANES)] = zeros
                return 0
            lax.fori_loop(0, nwords // LANES, zbody, 0, unroll=4)

        def key_level(src_v, nchunks, shift, himask, prefix, remaining):
            zero_hist(HB * LANES)

            def hbody(j, _):
                for k in range(4):
                    o = j * 64 + k * LANES
                    u = src_v[pl.ds(o, LANES)]
                    match = (u & himask) == prefix
                    bins = lax.shift_right_logical(u, shift) & np.int32(HB - 1)
                    plsc.addupdate_scatter(hist_v, [bins * LANES + lane], ones,
                                           mask=match)
                return 0

            lax.fori_loop(0, nchunks, hbody, 0)

            def fbody(c, carry):
                found, b, rem, tot = carry
                cc = (HB // LANES - 1) - c
                base = (cc * LANES + lane) * LANES
                acc = zeros
                for l in range(LANES):
                    acc = acc + plsc.load_gather(hist_v, [base + l])
                rv = lax.rev(acc, (0,))
                cs = plsc.cumsum(rv)
                suf = tot + cs
                cond = suf >= rem
                anyc = _scalar(plsc.all_reduce_population_count(cond))
                ffs = plsc.all_reduce_ffs(cond)
                hit = jnp.logical_and(jnp.logical_not(found), anyc > 0)
                b_new = cc * LANES + (LANES - 1) - _scalar(ffs)
                s_at = _at_lane(suf, ffs, lane)
                h_at = _at_lane(rv, ffs, lane)
                above = s_at - h_at
                b = jnp.where(hit, b_new, b)
                rem = jnp.where(hit, rem - above, rem)
                found = jnp.logical_or(found, anyc > 0)
                tot = tot + _at_lane(cs, np.int32(LANES - 1), lane)
                return found, b, rem, tot

            init = (np.bool_(False), np.int32(0), remaining, np.int32(0))
            _, b, rem, _ = lax.fori_loop(0, HB // LANES, fbody, init)
            return b, rem

        def tie_level(nchunks, nbits_shift, t_u, sel_hi, hi_shift, r):
            zero_hist(TIE_B * LANES)

            def hbody(j, _):
                for k in range(4):
                    o = j * 64 + k * LANES
                    u = cu_v[pl.ds(o, LANES)]
                    lidx = cx_v[pl.ds(o, LANES)]
                    bsel = lax.shift_right_logical(lidx, hi_shift)
                    match = (u == t_u) & (bsel == sel_hi)
                    bins = (lax.shift_right_logical(lidx, nbits_shift)
                            & np.int32(TIE_B - 1))
                    plsc.addupdate_scatter(hist_v, [bins * LANES + lane], ones,
                                           mask=match)
                return 0

            lax.fori_loop(0, nchunks, hbody, 0)

            def fbody(c, carry):
                found, b, rem, tot = carry
                base = (c * LANES + lane) * LANES
                acc = zeros
                for l in range(LANES):
                    acc = acc + plsc.load_gather(hist_v, [base + l])
                cs = plsc.cumsum(acc)
                cum = tot + cs
                cond = cum >= rem
                anyc = _scalar(plsc.all_reduce_population_count(cond))
                ffs = plsc.all_reduce_ffs(cond)
                hit = jnp.logical_and(jnp.logical_not(found), anyc > 0)
                b_new = c * LANES + _scalar(ffs)
                c_at = _at_lane(cum, ffs, lane)
                h_at = _at_lane(acc, ffs, lane)
                below = c_at - h_at
                b = jnp.where(hit, b_new, b)
                rem = jnp.where(hit, rem - below, rem)
                found = jnp.logical_or(found, anyc > 0)
                tot = tot + _at_lane(cs, np.int32(LANES - 1), lane)
                return found, b, rem, tot

            init = (np.bool_(False), np.int32(0), r, np.int32(0))
            _, b, rem, _ = lax.fori_loop(0, TIE_B // LANES, fbody, init)
            return b, rem

        def pad_cand(cnt):
            for k in range(4):
                cu_v[pl.ds(cnt + k * LANES, LANES)] = zeros

        def compact_from_keys(b0):
            def cbody(j, ptr):
                for k in range(4):
                    o = j * 64 + k * LANES
                    u = keys_v[pl.ds(o, LANES)]
                    m = lax.shift_right_logical(u, np.int32(24)) == b0
                    cs = plsc.cumsum(m.astype(jnp.int32))
                    idx = ptr + cs - 1
                    plsc.store_scatter(cu_v, [idx], u, mask=m)
                    plsc.store_scatter(cx_v, [idx], o + lane - off, mask=m)
                    ptr = ptr + _at_lane(cs, np.int32(LANES - 1), lane)
                return ptr

            cnt = lax.fori_loop(0, nv4, cbody, np.int32(0))
            pad_cand(cnt)
            return cnt

        def compact_cand(himask, prefix, nchunks):
            def cbody(j, ptr):
                for k in range(4):
                    o = j * 64 + k * LANES
                    u = cu_v[pl.ds(o, LANES)]
                    xv = cx_v[pl.ds(o, LANES)]
                    m = (u & himask) == prefix
                    cs = plsc.cumsum(m.astype(jnp.int32))
                    idx = ptr + cs - 1
                    plsc.store_scatter(cu_v, [idx], u, mask=m)
                    plsc.store_scatter(cx_v, [idx], xv, mask=m)
                    ptr = ptr + _at_lane(cs, np.int32(LANES - 1), lane)
                return ptr

            cnt = lax.fori_loop(0, nchunks, cbody, np.int32(0))
            pad_cand(cnt)
            return cnt

        def full_search(_):
            pltpu.sync_copy(att_hbm.at[pl.ds(a, WIN)], win_v)

            def conv_body(j, _):
                for k in range(4):
                    o = j * 64 + k * LANES
                    v = win_v[pl.ds(o, LANES)]
                    bits = lax.bitcast_convert_type(v, jnp.int32)
                    u = _sortable_i32(bits) ^ IMIN
                    lidx = o + lane - off
                    valid = (lidx >= 0) & (lidx < seglen)
                    keys_v[pl.ds(o, LANES)] = jnp.where(valid, u, 0)
                return 0

            lax.fori_loop(0, nv4, conv_body, 0)

            b0, rem = key_level(keys_v, nv4, np.int32(24), np.int32(0),
                                np.int32(0), np.int32(TOPK))
            prefix = lax.shift_left(b0, np.int32(24))
            cnt = compact_from_keys(b0)
            ncc = lax.shift_right_logical(cnt + np.int32(63), 6)
            himasks = {16: np.int32(-65536),
                       8: np.int32(-256),
                       0: np.int32(-1)}
            hiprev = np.int32(-16777216)
            for shift in (16, 8, 0):
                b, rem = key_level(cu_v, ncc, np.int32(shift), hiprev, prefix,
                                   rem)
                prefix = prefix | lax.shift_left(b, np.int32(shift))
                hiprev = himasks[shift]
                cnt = compact_cand(hiprev, prefix, ncc)
                ncc = lax.shift_right_logical(cnt + np.int32(63), 6)
            b0t, rem = tie_level(ncc, np.int32(6), prefix, np.int32(0),
                                 np.int32(12), rem)
            b1t, rem = tie_level(ncc, np.int32(0), prefix, b0t, np.int32(6),
                                 rem)
            tie = lax.shift_left(b0t, np.int32(6)) | b1t
            return prefix ^ IMIN, tie

        def trivial(_):
            return IMIN, np.int32(0)

        t_s, tie = lax.cond(seglen > TOPK, full_search, trivial, 0)

        res = jnp.where(lane == 0, t_s, jnp.where(lane == 1, tie, 0))
        res_v[...] = res
        pltpu.sync_copy(res_v, out_hbm.at[seg])


def _sc_thresholds(att_flat, starts, length):
    return _get_sc_kernel()(att_flat, starts, length)


def _pool_body(xt_ref, att_ref, starts_ref, lens_ref, thr_ref, tie_ref,
               lenf_ref, o_ref, acc_ref):
    blk = pl.program_id(0)
    nblk = pl.num_programs(0)

    @pl.when(blk == 0)
    def _():
        acc_ref[...] = jnp.zeros((NSEG, D), jnp.float32)

    rows = lax.broadcasted_iota(jnp.int32, (NSEG, BLK), 1) + blk * BLK
    bits = lax.bitcast_convert_type(att_ref[...].reshape(1, BLK),
                                    jnp.int32)
    key1 = _sortable_i32(bits)
    key = jnp.broadcast_to(key1, (NSEG, BLK))
    starts = starts_ref[...]
    lens = lens_ref[...]
    thr = thr_ref[...]
    tie = tie_ref[...]
    in_seg = (rows >= starts) & (rows < starts + lens)
    loc = rows - starts
    sel = (key > thr) | ((key == thr) & (loc <= tie))
    w = (in_seg & sel).astype(jnp.float32)
    acc_ref[...] += lax.dot_general(
        w, xt_ref[...], (((1,), (1,)), ((), ())),
        preferred_element_type=jnp.float32,
        precision=lax.Precision.HIGHEST)

    @pl.when(blk == nblk - 1)
    def _():
        res = acc_ref[...] / lenf_ref[...]
        nrm = jnp.sqrt(jnp.sum(res * res, axis=1, keepdims=True))
        o_ref[...] = res / jnp.maximum(nrm, 1e-12)


def _pool(xt, att, starts2d, lens2d, thr2d, tie2d, lenf2d):
    full = lambda i: (0, 0)
    return pl.pallas_call(
        _pool_body,
        grid=(NBLK,),
        in_specs=[
            pl.BlockSpec((D, BLK), lambda i: (0, i)),
            pl.BlockSpec((1, 1, BLK), lambda i: (i, 0, 0)),
            pl.BlockSpec((NSEG, 1), full),
            pl.BlockSpec((NSEG, 1), full),
            pl.BlockSpec((NSEG, 1), full),
            pl.BlockSpec((NSEG, 1), full),
            pl.BlockSpec((NSEG, 1), full),
        ],
        out_specs=pl.BlockSpec((NSEG, D), full),
        out_shape=jax.ShapeDtypeStruct((NSEG, D), jnp.float32),
        scratch_shapes=[pltpu.VMEM((NSEG, D), jnp.float32)],
    )(xt, att, starts2d, lens2d, thr2d, tie2d, lenf2d)


def kernel(x, length, conv_w, conv_b):
    xt = x.T
    w2d = conv_w[:, :, 0].astype(jnp.float32)
    b2d = conv_b.reshape(1, 1).astype(jnp.float32)
    length = length.astype(jnp.int32)
    starts = jnp.concatenate(
        [jnp.zeros((1,), jnp.int32), jnp.cumsum(length)[:-1]])

    att2d = _compute_att(xt, w2d, b2d)
    att_flat = att2d.reshape(N)

    sc_out = _sc_thresholds(att_flat, starts, length)
    thr2d = sc_out[:, 0:1]
    tie2d = sc_out[:, 1:2]

    return _pool(xt, att2d, starts.reshape(NSEG, 1), length.reshape(NSEG, 1),
                 thr2d, tie2d, length.astype(jnp.float32).reshape(NSEG, 1))


# speedup vs baseline: 15.2616x; 1.0800x over previous
import functools

import jax
import jax.numpy as jnp
import numpy as np
from jax import lax
from jax.experimental import pallas as pl
from jax.experimental.pallas import tpu as pltpu
from jax.experimental.pallas import tpu_sc as plsc

N = 65536
D = 32
NSEG = 16
TOPK = 1024
LANES = 16
WIN = 4160
NV = WIN // LANES
HB = 256
TIE_B = 64
NBLK = 8
BLK = N // NBLK
IMIN = np.int32(-2147483648)


def _sortable_i32(bits):
    return jnp.where(bits >= 0, bits, bits ^ np.int32(0x7FFFFFFF))


def _att_body(xt_ref, w_ref, b_ref, o_ref):
    xt = xt_ref[...]
    w = w_ref[...]
    att = lax.dot_general(w, xt, (((1,), (0,)), ((), ())),
                          preferred_element_type=jnp.float32)
    o_ref[...] = (att + b_ref[0, 0]).reshape(1, 1, BLK)


def _compute_att(xt, w2d, b2d):
    return pl.pallas_call(
        _att_body,
        grid=(NBLK,),
        in_specs=[
            pl.BlockSpec((D, BLK), lambda i: (0, i)),
            pl.BlockSpec((1, D), lambda i: (0, 0)),
            pl.BlockSpec((1, 1), lambda i: (0, 0)),
        ],
        out_specs=pl.BlockSpec((1, 1, BLK), lambda i: (i, 0, 0)),
        out_shape=jax.ShapeDtypeStruct((NBLK, 1, BLK), jnp.float32),
    )(xt, w2d, b2d)


SC_SCRATCH = (
    ((WIN,), jnp.float32),
    ((WIN,), jnp.int32),
    ((HB * LANES,), jnp.int32),
    ((WIN + 64,), jnp.int32),
    ((WIN + 64,), jnp.int32),
    ((LANES,), jnp.int32),
    ((LANES,), jnp.int32),
)


@functools.lru_cache(maxsize=None)
def _get_sc_kernel():
    mesh = plsc.VectorSubcoreMesh(core_axis_name="c", subcore_axis_name="s")
    return functools.partial(
        pl.kernel,
        out_type=jax.ShapeDtypeStruct((NSEG, LANES), jnp.int32),
        mesh=mesh,
        scratch_types=[pltpu.VMEM(s, d) for s, d in SC_SCRATCH],
        compiler_params=pltpu.CompilerParams(needs_layout_passes=False),
    )(_sc_body)


def _at_lane(v, pos, lane):
    return jnp.sum(jnp.where(lane == pos, v, 0), axis=0)


def _scalar(v):
    return jnp.max(v, axis=0)


def _sc_body(att_hbm, lens_hbm, out_hbm, win_v, keys_v, hist_v,
             cu_v, cx_v, lv_v, res_v):
    lane = lax.iota(jnp.int32, LANES)
    cid = lax.axis_index("c")
    sid = lax.axis_index("s")
    wid = sid * 2 + cid

    @pl.when(wid < NSEG)
    def _():
        seg = wid
        pltpu.sync_copy(lens_hbm, lv_v)
        lv = lv_v[...]
        starts_vec = plsc.cumsum(lv) - lv
        start = _at_lane(starts_vec, seg, lane)
        seglen = _at_lane(lv, seg, lane)

        a = pl.multiple_of(jnp.minimum(start & np.int32(-8), np.int32(N - WIN)), 8)
        off = start - a
        nv4 = lax.shift_right_logical(off + seglen + np.int32(63), 6)

        zeros = lane & 0
        ones = zeros + 1

        def zero_hist(nwords):
            def zbody(k, _):
                hist_v[pl.ds(k * LANES, LANES)] = zeros
                return 0
            lax.fori_loop(0, nwords // LANES, zbody, 0, unroll=4)

        def key_find(remaining):
            def fbody(c, carry):
                found, b, rem, tot = carry
                cc = (HB // LANES - 1) - c
                base = (cc * LANES + lane) * LANES
                acc = zeros
                for l in range(LANES):
                    acc = acc + plsc.load_gather(hist_v, [base + l])
                rv = lax.rev(acc, (0,))
                cs = plsc.cumsum(rv)
                suf = tot + cs
                cond = suf >= rem
                anyc = _scalar(plsc.all_reduce_population_count(cond))
                ffs = plsc.all_reduce_ffs(cond)
                hit = jnp.logical_and(jnp.logical_not(found), anyc > 0)
                b_new = cc * LANES + (LANES - 1) - _scalar(ffs)
                s_at = _at_lane(suf, ffs, lane)
                h_at = _at_lane(rv, ffs, lane)
                above = s_at - h_at
                b = jnp.where(hit, b_new, b)
                rem = jnp.where(hit, rem - above, rem)
                found = jnp.logical_or(found, anyc > 0)
                tot = tot + _at_lane(cs, np.int32(LANES - 1), lane)
                return found, b, rem, tot

            init = (np.bool_(False), np.int32(0), remaining, np.int32(0))
            _, b, rem, _ = lax.fori_loop(0, HB // LANES, fbody, init)
            return b, rem

        def key_level(src_v, nchunks, shift, himask, prefix, remaining):
            zero_hist(HB * LANES)

            def hbody(j, _):
                for k in range(4):
                    o = j * 64 + k * LANES
                    u = src_v[pl.ds(o, LANES)]
                    match = (u & himask) == prefix
                    bins = lax.shift_right_logical(u, shift) & np.int32(HB - 1)
                    plsc.addupdate_scatter(hist_v, [bins * LANES + lane], ones,
                                           mask=match)
                return 0

            lax.fori_loop(0, nchunks, hbody, 0)
            return key_find(remaining)

        def tie_level(nchunks, nbits_shift, t_u, sel_hi, hi_shift, r):
            zero_hist(TIE_B * LANES)

            def hbody(j, _):
                for k in range(4):
                    o = j * 64 + k * LANES
                    u = cu_v[pl.ds(o, LANES)]
                    lidx = cx_v[pl.ds(o, LANES)]
                    bsel = lax.shift_right_logical(lidx, hi_shift)
                    match = (u == t_u) & (bsel == sel_hi)
                    bins = (lax.shift_right_logical(lidx, nbits_shift)
                            & np.int32(TIE_B - 1))
                    plsc.addupdate_scatter(hist_v, [bins * LANES + lane], ones,
                                           mask=match)
                return 0

            lax.fori_loop(0, nchunks, hbody, 0)

            def fbody(c, carry):
                found, b, rem, tot = carry
                base = (c * LANES + lane) * LANES
                acc = zeros
                for l in range(LANES):
                    acc = acc + plsc.load_gather(hist_v, [base + l])
                cs = plsc.cumsum(acc)
                cum = tot + cs
                cond = cum >= rem
                anyc = _scalar(plsc.all_reduce_population_count(cond))
                ffs = plsc.all_reduce_ffs(cond)
                hit = jnp.logical_and(jnp.logical_not(found), anyc > 0)
                b_new = c * LANES + _scalar(ffs)
                c_at = _at_lane(cum, ffs, lane)
                h_at = _at_lane(acc, ffs, lane)
                below = c_at - h_at
                b = jnp.where(hit, b_new, b)
                rem = jnp.where(hit, rem - below, rem)
                found = jnp.logical_or(found, anyc > 0)
                tot = tot + _at_lane(cs, np.int32(LANES - 1), lane)
                return found, b, rem, tot

            init = (np.bool_(False), np.int32(0), r, np.int32(0))
            _, b, rem, _ = lax.fori_loop(0, TIE_B // LANES, fbody, init)
            return b, rem

        def pad_cand(cnt):
            for k in range(4):
                cu_v[pl.ds(cnt + k * LANES, LANES)] = zeros

        def compact_from_keys(b0):
            def cbody(j, ptr):
                for k in range(4):
                    o = j * 64 + k * LANES
                    u = keys_v[pl.ds(o, LANES)]
                    m = lax.shift_right_logical(u, np.int32(24)) == b0
                    cs = plsc.cumsum(m.astype(jnp.int32))
                    idx = ptr + cs - 1
                    plsc.store_scatter(cu_v, [idx], u, mask=m)
                    plsc.store_scatter(cx_v, [idx], o + lane - off, mask=m)
                    ptr = ptr + _at_lane(cs, np.int32(LANES - 1), lane)
                return ptr

            cnt = lax.fori_loop(0, nv4, cbody, np.int32(0))
            pad_cand(cnt)
            return cnt

        def compact_cand(himask, prefix, nchunks):
            def cbody(j, ptr):
                for k in range(4):
                    o = j * 64 + k * LANES
                    u = cu_v[pl.ds(o, LANES)]
                    xv = cx_v[pl.ds(o, LANES)]
                    m = (u & himask) == prefix
                    cs = plsc.cumsum(m.astype(jnp.int32))
                    idx = ptr + cs - 1
                    plsc.store_scatter(cu_v, [idx], u, mask=m)
                    plsc.store_scatter(cx_v, [idx], xv, mask=m)
                    ptr = ptr + _at_lane(cs, np.int32(LANES - 1), lane)
                return ptr

            cnt = lax.fori_loop(0, nchunks, cbody, np.int32(0))
            pad_cand(cnt)
            return cnt

        def full_search(_):
            zero_hist(HB * LANES)
            pltpu.sync_copy(att_hbm.at[pl.ds(a, WIN)], win_v)

            def conv_hist_body(j, _):
                for k in range(4):
                    o = j * 64 + k * LANES
                    v = win_v[pl.ds(o, LANES)]
                    bits = lax.bitcast_convert_type(v, jnp.int32)
                    u = _sortable_i32(bits) ^ IMIN
                    lidx = o + lane - off
                    valid = (lidx >= 0) & (lidx < seglen)
                    u = jnp.where(valid, u, 0)
                    keys_v[pl.ds(o, LANES)] = u
                    bins = lax.shift_right_logical(u, np.int32(24))
                    plsc.addupdate_scatter(hist_v, [bins * LANES + lane], ones,
                                           mask=valid)
                return 0

            lax.fori_loop(0, nv4, conv_hist_body, 0)

            b0, rem = key_find(np.int32(TOPK))
            prefix = lax.shift_left(b0, np.int32(24))
            cnt = compact_from_keys(b0)
            ncc = lax.shift_right_logical(cnt + np.int32(63), 6)
            himasks = {16: np.int32(-65536),
                       8: np.int32(-256),
                       0: np.int32(-1)}
            hiprev = np.int32(-16777216)
            for shift in (16, 8, 0):
                b, rem = key_level(cu_v, ncc, np.int32(shift), hiprev, prefix,
                                   rem)
                prefix = prefix | lax.shift_left(b, np.int32(shift))
                hiprev = himasks[shift]
                cnt = compact_cand(hiprev, prefix, ncc)
                ncc = lax.shift_right_logical(cnt + np.int32(63), 6)
            def do_ties(_):
                b0t, r2 = tie_level(ncc, np.int32(6), prefix, np.int32(0),
                                    np.int32(12), rem)
                b1t, _ = tie_level(ncc, np.int32(0), prefix, b0t, np.int32(6),
                                   r2)
                return lax.shift_left(b0t, np.int32(6)) | b1t

            tie = lax.cond(cnt == rem, lambda _: np.int32(4095), do_ties, 0)
            return prefix ^ IMIN, tie

        def trivial(_):
            return IMIN, np.int32(0)

        t_s, tie = lax.cond(seglen > TOPK, full_search, trivial, 0)

        res = jnp.where(lane == 0, t_s, jnp.where(lane == 1, tie, 0))
        res_v[...] = res
        pltpu.sync_copy(res_v, out_hbm.at[seg])


def _sc_thresholds(att_flat, length):
    return _get_sc_kernel()(att_flat, length)


def _pool_body(xt_ref, att_ref, lens_ref, sc_ref, o_ref, acc_ref):
    blk = pl.program_id(0)
    nblk = pl.num_programs(0)

    @pl.when(blk == 0)
    def _():
        acc_ref[...] = jnp.zeros((NSEG, D), jnp.float32)

    lens = lens_ref[...]
    lensf = lens.astype(jnp.float32)
    ii = lax.broadcasted_iota(jnp.int32, (NSEG, NSEG), 0)
    jj = lax.broadcasted_iota(jnp.int32, (NSEG, NSEG), 1)
    tril = (jj < ii).astype(jnp.float32)
    starts = lax.dot_general(
        tril, lensf, (((1,), (0,)), ((), ())),
        preferred_element_type=jnp.float32,
        precision=lax.Precision.HIGHEST).astype(jnp.int32)
    scv = sc_ref[...]
    thr = lax.slice(scv, (0, 0), (NSEG, 1))
    tie = lax.slice(scv, (0, 1), (NSEG, 2))

    rows = lax.broadcasted_iota(jnp.int32, (NSEG, BLK), 1) + blk * BLK
    bits = lax.bitcast_convert_type(att_ref[...].reshape(1, BLK),
                                    jnp.int32)
    key1 = _sortable_i32(bits)
    key = jnp.broadcast_to(key1, (NSEG, BLK))
    in_seg = (rows >= starts) & (rows < starts + lens)
    loc = rows - starts
    sel = (key > thr) | ((key == thr) & (loc <= tie))
    w = (in_seg & sel).astype(jnp.float32)
    acc_ref[...] += lax.dot_general(
        w, xt_ref[...], (((1,), (1,)), ((), ())),
        preferred_element_type=jnp.float32,
        precision=lax.Precision.HIGHEST)

    @pl.when(blk == nblk - 1)
    def _():
        res = acc_ref[...] / lensf
        nrm = jnp.sqrt(jnp.sum(res * res, axis=1, keepdims=True))
        o_ref[...] = res / jnp.maximum(nrm, 1e-12)


def _pool(xt, att, lens2d, sc_out):
    full = lambda i: (0, 0)
    return pl.pallas_call(
        _pool_body,
        grid=(NBLK,),
        in_specs=[
            pl.BlockSpec((D, BLK), lambda i: (0, i)),
            pl.BlockSpec((1, 1, BLK), lambda i: (i, 0, 0)),
            pl.BlockSpec((NSEG, 1), full),
            pl.BlockSpec((NSEG, LANES), full),
        ],
        out_specs=pl.BlockSpec((NSEG, D), full),
        out_shape=jax.ShapeDtypeStruct((NSEG, D), jnp.float32),
        scratch_shapes=[pltpu.VMEM((NSEG, D), jnp.float32)],
    )(xt, att, lens2d, sc_out)


def kernel(x, length, conv_w, conv_b):
    xt = x.T
    w2d = conv_w[:, :, 0].astype(jnp.float32)
    b2d = conv_b.reshape(1, 1).astype(jnp.float32)
    length = length.astype(jnp.int32)

    att2d = _compute_att(xt, w2d, b2d)
    att_flat = att2d.reshape(N)

    sc_out = _sc_thresholds(att_flat, length)

    return _pool(xt, att2d, length.reshape(NSEG, 1), sc_out)
